# Initial kernel scaffold; baseline (speedup 1.0000x reference)
import jax
import jax.numpy as jnp
from jax.experimental import pallas as pl


def kernel(input, heads, W_dt, U_dt, b_dt, W_td, U_td, b_td):
    raise NotImplementedError("write your pallas kernel here")


# speedup vs baseline: 8.9115x; 8.9115x over previous
import functools

import jax
import jax.numpy as jnp
from jax.experimental import pallas as pl
from jax.experimental.pallas import tpu as pltpu


def _gru_math(gx, gh, h_prev, H):
    r = jax.nn.sigmoid(gx[:, :H] + gh[:, :H])
    z = jax.nn.sigmoid(gx[:, H:2 * H] + gh[:, H:2 * H])
    n = jnp.tanh(gx[:, 2 * H:] + r * gh[:, 2 * H:])
    return (1.0 - z) * n + z * h_prev


def _dt_kernel(heads_ref, emb_ref, w_ref, u_ref, b_ref, hid_ref, cs_ref):
    L, B, H = cs_ref.shape
    i = pl.program_id(0)
    t = L - 1 - i

    @pl.when(i == 0)
    def _():
        cs_ref[...] = jnp.zeros_like(cs_ref)

    h_prev = cs_ref[pl.ds(t, 1)].reshape(B, H)
    x = emb_ref[0]
    gx = jnp.dot(x, w_ref[...], preferred_element_type=jnp.float32) + b_ref[...]
    gh = jnp.dot(h_prev, u_ref[...], preferred_element_type=jnp.float32)
    h = _gru_math(gx, gh, h_prev, H)
    hid_ref[0] = h

    @pl.when(i != L - 1)
    def _():
        def body(b, _):
            p = heads_ref[b, t]
            row = hid_ref[0, pl.ds(b, 1), :]
            cs_ref[pl.ds(p, 1), pl.ds(b, 1), :] += row.reshape(1, 1, H)
            return 0

        jax.lax.fori_loop(0, B, body, 0)


def _td_kernel(heads_ref, emb_ref, w_ref, u_ref, b_ref, hid_ref,
               keep_ref, hpar_ref):
    L, B, H = keep_ref.shape
    t = pl.program_id(0)

    @pl.when(t == 0)
    def _():
        hpar_ref[...] = jnp.zeros_like(hpar_ref)

    @pl.when(t != 0)
    def _():
        def body(b, _):
            p = heads_ref[b, t]
            hpar_ref[pl.ds(b, 1), :] = keep_ref[pl.ds(p, 1), pl.ds(b, 1), :
                                                ].reshape(1, H)
            return 0

        jax.lax.fori_loop(0, B, body, 0)

    h_par = hpar_ref[...]
    x = emb_ref[0]
    gx = jnp.dot(x, w_ref[...], preferred_element_type=jnp.float32) + b_ref[...]
    gh = jnp.dot(h_par, u_ref[...], preferred_element_type=jnp.float32)
    h = _gru_math(gx, gh, h_par, H)
    hid_ref[0] = h
    keep_ref[pl.ds(t, 1)] = h.reshape(1, B, H)


def _run_scan(which, emb, heads, W, U, b):
    L, B, D = emb.shape
    H = U.shape[0]
    b2 = b.reshape(1, 3 * H)
    if which == "dt":
        body = _dt_kernel
        node = lambda i: L - 1 - i
        scratch = [pltpu.VMEM((L, B, H), jnp.float32)]
    else:
        body = _td_kernel
        node = lambda i: i
        scratch = [pltpu.VMEM((L, B, H), jnp.float32),
                   pltpu.VMEM((B, H), jnp.float32)]

    grid_spec = pltpu.PrefetchScalarGridSpec(
        num_scalar_prefetch=1,
        grid=(L,),
        in_specs=[
            pl.BlockSpec((1, B, D), lambda i, h_ref: (node(i), 0, 0)),
            pl.BlockSpec((D, 3 * H), lambda i, h_ref: (0, 0)),
            pl.BlockSpec((H, 3 * H), lambda i, h_ref: (0, 0)),
            pl.BlockSpec((1, 3 * H), lambda i, h_ref: (0, 0)),
        ],
        out_specs=pl.BlockSpec((1, B, H), lambda i, h_ref: (node(i), 0, 0)),
        scratch_shapes=scratch,
    )
    return pl.pallas_call(
        body,
        grid_spec=grid_spec,
        out_shape=jax.ShapeDtypeStruct((L, B, H), jnp.float32),
        compiler_params=pltpu.CompilerParams(
            dimension_semantics=("arbitrary",),
        ),
    )(heads, emb, W, U, b2)


@jax.jit
def kernel(input, heads, W_dt, U_dt, b_dt, W_td, U_td, b_td):
    L, B, D = input.shape
    H = U_dt.shape[0]
    dt_hid = _run_scan("dt", input, heads, W_dt, U_dt, b_dt)
    td_hid = _run_scan("td", input, heads, W_td, U_td, b_td)
    outputs = jnp.concatenate([dt_hid, td_hid], axis=2).transpose(1, 0, 2)
    output_t = jnp.concatenate([dt_hid[0], td_hid[L - 1]], axis=1)[None]
    return outputs, output_t


# speedup vs baseline: 9.5635x; 1.0732x over previous
import jax
import jax.numpy as jnp
from jax.experimental import pallas as pl
from jax.experimental.pallas import tpu as pltpu


def _gru_math(gx, gh, h_prev, H):
    r = jax.nn.sigmoid(gx[:, :H] + gh[:, :H])
    z = jax.nn.sigmoid(gx[:, H:2 * H] + gh[:, H:2 * H])
    n = jnp.tanh(gx[:, 2 * H:] + r * gh[:, 2 * H:])
    return (1.0 - z) * n + z * h_prev


def _gx_kernel(x_ref, w_ref, b_ref, o_ref):
    o_ref[...] = (
        jnp.dot(x_ref[...], w_ref[...], preferred_element_type=jnp.float32)
        + b_ref[...]
    )


def _precompute_gx(emb, W, b):
    LB, D = emb.shape
    block_m = min(1024, LB)
    N = W.shape[1]
    return pl.pallas_call(
        _gx_kernel,
        grid=(LB // block_m,),
        in_specs=[
            pl.BlockSpec((block_m, D), lambda i: (i, 0)),
            pl.BlockSpec((D, N), lambda i: (0, 0)),
            pl.BlockSpec((1, N), lambda i: (0, 0)),
        ],
        out_specs=pl.BlockSpec((block_m, N), lambda i: (i, 0)),
        out_shape=jax.ShapeDtypeStruct((LB, N), jnp.float32),
        compiler_params=pltpu.CompilerParams(
            dimension_semantics=("arbitrary",),
        ),
    )(emb, W, b)


def _dt_kernel(heads_ref, gx_ref, u_ref, hid_ref, cs_ref):
    L, B, H = cs_ref.shape
    i = pl.program_id(0)
    t = L - 1 - i

    @pl.when(i == 0)
    def _():
        cs_ref[...] = jnp.zeros_like(cs_ref)

    h_prev = cs_ref[pl.ds(t, 1)].reshape(B, H)
    gx = gx_ref[0]
    gh = jnp.dot(h_prev, u_ref[...], preferred_element_type=jnp.float32)
    h = _gru_math(gx, gh, h_prev, H)
    hid_ref[0] = h

    @pl.when(i != L - 1)
    def _():
        def body(b, _):
            p = heads_ref[b, t]
            row = hid_ref[0, pl.ds(b, 1), :]
            cs_ref[pl.ds(p, 1), pl.ds(b, 1), :] += row.reshape(1, 1, H)
            return 0

        jax.lax.fori_loop(0, B, body, 0)


def _td_kernel(heads_ref, gx_ref, u_ref, hid_ref, keep_ref, hpar_ref):
    L, B, H = keep_ref.shape
    t = pl.program_id(0)

    @pl.when(t == 0)
    def _():
        hpar_ref[...] = jnp.zeros_like(hpar_ref)

    @pl.when(t != 0)
    def _():
        def body(b, _):
            p = heads_ref[b, t]
            hpar_ref[pl.ds(b, 1), :] = keep_ref[pl.ds(p, 1), pl.ds(b, 1), :
                                                ].reshape(1, H)
            return 0

        jax.lax.fori_loop(0, B, body, 0)

    h_par = hpar_ref[...]
    gx = gx_ref[0]
    gh = jnp.dot(h_par, u_ref[...], preferred_element_type=jnp.float32)
    h = _gru_math(gx, gh, h_par, H)
    hid_ref[0] = h
    keep_ref[pl.ds(t, 1)] = h.reshape(1, B, H)


def _run_scan(which, gx, heads, U):
    L, B, _ = gx.shape
    H = U.shape[0]
    if which == "dt":
        body = _dt_kernel
        node = lambda i: L - 1 - i
        col = 0
        scratch = [pltpu.VMEM((L, B, H), jnp.float32)]
    else:
        body = _td_kernel
        node = lambda i: i
        col = 1
        scratch = [pltpu.VMEM((L, B, H), jnp.float32),
                   pltpu.VMEM((B, H), jnp.float32)]

    grid_spec = pltpu.PrefetchScalarGridSpec(
        num_scalar_prefetch=1,
        grid=(L,),
        in_specs=[
            pl.BlockSpec((1, B, 3 * H), lambda i, h_ref: (node(i), 0, col)),
            pl.BlockSpec((H, 3 * H), lambda i, h_ref: (0, 0)),
        ],
        out_specs=pl.BlockSpec((1, B, H), lambda i, h_ref: (node(i), 0, 0)),
        scratch_shapes=scratch,
    )
    return pl.pallas_call(
        body,
        grid_spec=grid_spec,
        out_shape=jax.ShapeDtypeStruct((L, B, H), jnp.float32),
        compiler_params=pltpu.CompilerParams(
            dimension_semantics=("arbitrary",),
        ),
    )(heads, gx, U)


@jax.jit
def kernel(input, heads, W_dt, U_dt, b_dt, W_td, U_td, b_td):
    L, B, D = input.shape
    H = U_dt.shape[0]
    W = jnp.concatenate([W_dt, W_td], axis=1)
    b = jnp.concatenate([b_dt, b_td])[None]
    gx = _precompute_gx(input.reshape(L * B, D), W, b)
    gx = gx.reshape(L, B, 6 * H)
    dt_hid = _run_scan("dt", gx, heads, U_dt)
    td_hid = _run_scan("td", gx, heads, U_td)
    outputs = jnp.concatenate([dt_hid, td_hid], axis=2).transpose(1, 0, 2)
    output_t = jnp.concatenate([dt_hid[0], td_hid[L - 1]], axis=1)[None]
    return outputs, output_t
